# baseline (speedup 1.0000x reference)
import functools

import jax
import jax.numpy as jnp
from jax import lax
from jax.experimental import pallas as pl
from jax.experimental.pallas import tpu as pltpu
from jax.experimental.pallas import tpu_sc as plsc

NC = 2
NS = 16
LANES = 16
NW = NC * NS
UNROLL = 10

SENTINEL = -3.0e38
THRESH = -1.0e37


def _tc1_body(xt_ref, a_ref, wb_ref, b_ref, pbt_ref, qt_ref):
    xt = xt_ref[...]
    d = xt.shape[0]
    dn0 = (((0,), (0,)), ((), ()))
    pbt = lax.dot_general(a_ref[...], xt, dn0,
                          preferred_element_type=jnp.float32,
                          precision=lax.Precision.HIGHEST)
    pbt_ref[...] = pbt + b_ref[...].reshape(d, 1)
    qt_ref[...] = lax.dot_general(wb_ref[...], xt, dn0,
                                  preferred_element_type=jnp.float32,
                                  precision=lax.Precision.HIGHEST)


def _pack_body(ei_ref, pk_ref):
    ei = ei_ref[...]
    s = ei[0].astype(jnp.uint32)
    dd = ei[1].astype(jnp.uint32)
    pk_ref[...] = (dd << 16) | s


def _extract(pk):
    sv = (pk & jnp.uint32(0xFFFF)).astype(jnp.int32)
    dv = (pk >> 16).astype(jnp.int32)
    return sv, dv


def _sc_body(n, e, d, ch, qt_hbm, pk_hbm, part_hbm,
             qcol, acc, pbuf0, pbuf1, sem0, sem1, qsem, asem):
    epw = e // NW
    nch = epw // ch
    cid = lax.axis_index("c")
    sid = lax.axis_index("s")
    w = sid * NC + cid
    e0 = w * epw
    nu = ch // (LANES * UNROLL)
    sent16 = jnp.full((LANES,), SENTINEL, jnp.float32)
    zero16 = jnp.zeros((LANES,), jnp.int32)
    one16 = jnp.ones((LANES,), jnp.int32)
    bufs = ((pbuf0, sem0), (pbuf1, sem1))

    def chunk_src(c):
        return pk_hbm.at[pl.ds(e0 + c * ch, ch)]

    def acc_dst(col):
        return part_hbm.at[pl.ds((w * d + col) * n, n)]

    def process(pbuf, c):
        def blk_body(i, vacc):
            base = i * (LANES * UNROLL)
            dvs, qs = [], []
            for j in range(UNROLL):
                pk = pbuf[pl.ds(base + j * LANES, LANES)]
                sv, dv = _extract(pk)
                dvs.append(dv)
                qs.append(plsc.load_gather(qcol, [sv]))
            avs = [plsc.load_gather(acc, [dv]) for dv in dvs]
            for j in range(UNROLL):
                plsc.store_scatter(acc, [dvs[j]], qs[j],
                                   mask=qs[j] > avs[j])
            a1s = [plsc.load_gather(acc, [dv]) for dv in dvs]
            for j in range(UNROLL):
                plsc.store_scatter(acc, [dvs[j]], qs[j],
                                   mask=qs[j] > a1s[j])
            viol = None
            for j in range(UNROLL):
                a2 = plsc.load_gather(acc, [dvs[j]])
                v = qs[j] > a2
                viol = v if viol is None else (viol | v)
            return vacc | jnp.where(viol, one16, zero16)

        vacc = lax.fori_loop(0, nu, blk_body, zero16)

        @pl.when(jnp.max(vacc) > 0)
        def _deep_repair():
            def grp(g, c2):
                pk = pbuf[pl.ds(g * LANES, LANES)]
                sv, dv = _extract(pk)
                q = plsc.load_gather(qcol, [sv])

                def rnd(r, c3):
                    a = plsc.load_gather(acc, [dv])
                    plsc.store_scatter(acc, [dv], q, mask=q > a)
                    return c3
                lax.fori_loop(0, LANES, rnd, 0)
                return c2
            lax.fori_loop(0, ch // LANES, grp, 0)

    for col in range(d):
        pltpu.async_copy(qt_hbm.at[pl.ds(col * n, n)], qcol, qsem)
        pltpu.async_copy(chunk_src(0), pbuf0, sem0)
        if col > 0:
            pltpu.make_async_copy(acc, acc_dst(col - 1), asem).wait()
        pltpu.make_async_copy(qt_hbm.at[pl.ds(col * n, n)], qcol, qsem).wait()

        def init_body(i, carry):
            for j in range(8):
                acc[pl.ds((i * 8 + j) * LANES, LANES)] = sent16
            return carry
        lax.fori_loop(0, n // (LANES * 8), init_body, 0)

        def pair_body(k, carry):
            for bsel in range(2):
                pbuf, sem = bufs[bsel]
                nxt, nsem = bufs[1 - bsel]
                c = k * 2 + bsel
                pltpu.make_async_copy(chunk_src(c), pbuf, sem).wait()

                @pl.when(c < nch - 1)
                def _prefetch():
                    pltpu.async_copy(chunk_src(c + 1), nxt, nsem)

                process(pbuf, c)
            return carry
        lax.fori_loop(0, nch // 2, pair_body, 0)

        if nch % 2:
            pltpu.make_async_copy(chunk_src(nch - 1), pbuf0, sem0).wait()
            process(pbuf0, nch - 1)

        pltpu.async_copy(acc, acc_dst(col), asem)
    pltpu.make_async_copy(acc, acc_dst(d - 1), asem).wait()


def _tc2_body(part_ref, pbt_ref, out_ref):
    m = jnp.max(part_ref[...], axis=0)
    out_ref[...] = jnp.where(m > THRESH, m + pbt_ref[...], 0.0)


def kernel(x, edge_index, edge_f, edge_attr, W, b):
    del edge_f, edge_attr
    n, d = x.shape
    e = edge_index.shape[1]
    npad = ((n + 127) // 128) * 128
    assert n < (1 << 16) and npad % (LANES * 8) == 0 and e % NW == 0
    epw = e // NW
    ch = 4000 if epw % 4000 == 0 else epw
    assert ch % (LANES * UNROLL) == 0 and epw % ch == 0
    units = npad // 128
    u = max(v for v in range(1, min(units, 32) + 1) if units % v == 0)
    nb = u * 128
    eu = e // 128
    ue = max(v for v in range(1, min(eu, 1024) + 1) if eu % v == 0)
    eb = ue * 128

    edge_index = edge_index.astype(jnp.int32)
    a_mat = W[:d] - W[d:]
    wb_mat = W[d:]
    xtp = jnp.pad(x.T, ((0, 0), (0, npad - n)))

    pbt, qt = pl.pallas_call(
        _tc1_body,
        grid=(npad // nb,),
        in_specs=[
            pl.BlockSpec((d, nb), lambda i: (0, i)),
            pl.BlockSpec((d, d), lambda i: (0, 0)),
            pl.BlockSpec((d, d), lambda i: (0, 0)),
            pl.BlockSpec((d,), lambda i: (0,)),
        ],
        out_specs=[
            pl.BlockSpec((d, nb), lambda i: (0, i)),
            pl.BlockSpec((d, nb), lambda i: (0, i)),
        ],
        out_shape=[
            jax.ShapeDtypeStruct((d, npad), jnp.float32),
            jax.ShapeDtypeStruct((d, npad), jnp.float32),
        ],
    )(xtp, a_mat, wb_mat, b)

    packed = pl.pallas_call(
        _pack_body,
        grid=(e // eb,),
        in_specs=[pl.BlockSpec((2, eb), lambda i: (0, i))],
        out_specs=pl.BlockSpec((eb,), lambda i: (i,)),
        out_shape=jax.ShapeDtypeStruct((e,), jnp.uint32),
    )(edge_index)

    mesh = plsc.VectorSubcoreMesh(
        core_axis_name="c", subcore_axis_name="s",
        num_cores=NC, num_subcores=NS)
    partial = pl.kernel(
        functools.partial(_sc_body, npad, e, d, ch),
        out_type=jax.ShapeDtypeStruct((NW * d * npad,), jnp.float32),
        mesh=mesh,
        compiler_params=pltpu.CompilerParams(needs_layout_passes=False),
        scratch_types=[
            pltpu.VMEM((npad,), jnp.float32),
            pltpu.VMEM((npad,), jnp.float32),
            pltpu.VMEM((ch,), jnp.uint32),
            pltpu.VMEM((ch,), jnp.uint32),
            pltpu.SemaphoreType.DMA,
            pltpu.SemaphoreType.DMA,
            pltpu.SemaphoreType.DMA,
            pltpu.SemaphoreType.DMA,
        ],
    )(qt.reshape(-1), packed)
    partial = partial.reshape(NW, d, npad)

    outt = pl.pallas_call(
        _tc2_body,
        grid=(npad // nb,),
        in_specs=[
            pl.BlockSpec((NW, d, nb), lambda i: (0, 0, i)),
            pl.BlockSpec((d, nb), lambda i: (0, i)),
        ],
        out_specs=pl.BlockSpec((d, nb), lambda i: (0, i)),
        out_shape=jax.ShapeDtypeStruct((d, npad), jnp.float32),
    )(partial, pbt)
    return outt[:, :n].T


# speedup vs baseline: 1.0929x; 1.0929x over previous
import functools

import jax
import jax.numpy as jnp
import numpy as np
from jax import lax
from jax.experimental import pallas as pl
from jax.experimental.pallas import tpu as pltpu
from jax.experimental.pallas import tpu_sc as plsc

NC = 2
NS = 16
LANES = 16
NW = NC * NS
UNROLL = 10

SENTINEL = -3.0e38
THRESH = -1.0e37
HI_MASK = -65536
LO_MASK = 0xFFFF

_F32_BITS = int(np.float32(SENTINEL).view(np.uint32))


def _bf16_bits(u):
    return ((u + 0x7FFF + ((u >> 16) & 1)) >> 16) & 0xFFFF


_SENT_BF = _bf16_bits(_F32_BITS)
_SENT_PACKED = np.int32(np.uint32((_SENT_BF << 16) | _SENT_BF)).item()
_SENT_F32 = np.int32(np.uint32(_F32_BITS)).item()


def _tc1_body(xt_ref, a_ref, wb_ref, b_ref, pbt_ref, qt_ref):
    xt = xt_ref[...]
    d = xt.shape[0]
    dn0 = (((0,), (0,)), ((), ()))
    pbt = lax.dot_general(a_ref[...], xt, dn0,
                          preferred_element_type=jnp.float32,
                          precision=lax.Precision.HIGHEST)
    pbt_ref[...] = pbt + b_ref[...].reshape(d, 1)
    q = lax.dot_general(wb_ref[...], xt, dn0,
                        preferred_element_type=jnp.float32,
                        precision=lax.Precision.HIGHEST)
    qb = lax.bitcast_convert_type(q.astype(jnp.bfloat16),
                                  jnp.uint16).astype(jnp.uint32)
    rows = []
    for p in range(d // 2):
        rows.append(((qb[2 * p + 1] << 16) | qb[2 * p])[None])
    if d % 2:
        rows.append(lax.bitcast_convert_type(q[d - 1], jnp.uint32)[None])
    qt_ref[...] = jnp.concatenate(rows, axis=0).astype(jnp.int32)


def _pack_body(ei_ref, pk_ref):
    ei = ei_ref[...]
    s = ei[0].astype(jnp.uint32)
    dd = ei[1].astype(jnp.uint32)
    pk_ref[...] = ((dd << 16) | s).astype(jnp.int32)


def _extract(pk):
    sv = pk & LO_MASK
    dv = lax.shift_right_logical(pk, 16)
    return sv, dv


def _pend_val(qw, aw, packed):
    if packed:
        q0 = lax.bitcast_convert_type(qw << 16, jnp.float32)
        q1 = lax.bitcast_convert_type(qw & HI_MASK, jnp.float32)
        a0 = lax.bitcast_convert_type(aw << 16, jnp.float32)
        a1 = lax.bitcast_convert_type(aw & HI_MASK, jnp.float32)
        m0 = q0 > a0
        m1 = q1 > a1
        val = (jnp.where(m1, qw, aw) & HI_MASK) | \
              (jnp.where(m0, qw, aw) & LO_MASK)
        return m0 | m1, val
    q = lax.bitcast_convert_type(qw, jnp.float32)
    a = lax.bitcast_convert_type(aw, jnp.float32)
    return q > a, qw


def _sc_body(n, e, rows, ch, qt_hbm, pk_hbm, part_hbm,
             qcol, acc, pbuf0, pbuf1, sem0, sem1, qsem, asem):
    epw = e // NW
    nch = epw // ch
    cid = lax.axis_index("c")
    sid = lax.axis_index("s")
    w = sid * NC + cid
    e0 = w * epw
    nu = ch // (LANES * UNROLL)
    zero16 = jnp.zeros((LANES,), jnp.int32)
    one16 = jnp.ones((LANES,), jnp.int32)
    bufs = ((pbuf0, sem0), (pbuf1, sem1))

    def chunk_src(c):
        return pk_hbm.at[pl.ds(e0 + c * ch, ch)]

    def acc_dst(col):
        return part_hbm.at[pl.ds((w * len(rows) + col) * n, n)]

    def process(pbuf, packed):
        def blk_body(i, vacc):
            base = i * (LANES * UNROLL)
            dvs, qws = [], []
            for j in range(UNROLL):
                pk = pbuf[pl.ds(base + j * LANES, LANES)]
                sv, dv = _extract(pk)
                dvs.append(dv)
                qws.append(plsc.load_gather(qcol, [sv]))
            aws = [plsc.load_gather(acc, [dv]) for dv in dvs]
            for j in range(UNROLL):
                pend, val = _pend_val(qws[j], aws[j], packed)
                plsc.store_scatter(acc, [dvs[j]], val, mask=pend)
            a1s = [plsc.load_gather(acc, [dv]) for dv in dvs]
            for j in range(UNROLL):
                pend, val = _pend_val(qws[j], a1s[j], packed)
                plsc.store_scatter(acc, [dvs[j]], val, mask=pend)
            viol = None
            for j in range(UNROLL):
                a2 = plsc.load_gather(acc, [dvs[j]])
                v, _ = _pend_val(qws[j], a2, packed)
                viol = v if viol is None else (viol | v)
            return vacc | jnp.where(viol, one16, zero16)

        vacc = lax.fori_loop(0, nu, blk_body, zero16)

        @pl.when(jnp.max(vacc) > 0)
        def _deep_repair():
            def grp(g, c2):
                pk = pbuf[pl.ds(g * LANES, LANES)]
                sv, dv = _extract(pk)
                qw = plsc.load_gather(qcol, [sv])

                def rnd(r, c3):
                    aw = plsc.load_gather(acc, [dv])
                    pend, val = _pend_val(qw, aw, packed)
                    plsc.store_scatter(acc, [dv], val, mask=pend)
                    return c3
                lax.fori_loop(0, LANES, rnd, 0)
                return c2
            lax.fori_loop(0, ch // LANES, grp, 0)

    for col, (packed, sent_word) in enumerate(rows):
        pltpu.async_copy(qt_hbm.at[pl.ds(col * n, n)], qcol, qsem)
        pltpu.async_copy(chunk_src(0), pbuf0, sem0)
        if col > 0:
            pltpu.make_async_copy(acc, acc_dst(col - 1), asem).wait()
        pltpu.make_async_copy(qt_hbm.at[pl.ds(col * n, n)], qcol, qsem).wait()

        sent16 = jnp.full((LANES,), sent_word, jnp.int32)

        def init_body(i, carry):
            for j in range(8):
                acc[pl.ds((i * 8 + j) * LANES, LANES)] = sent16
            return carry
        lax.fori_loop(0, n // (LANES * 8), init_body, 0)

        def pair_body(k, carry):
            for bsel in range(2):
                pbuf, sem = bufs[bsel]
                nxt, nsem = bufs[1 - bsel]
                c = k * 2 + bsel
                pltpu.make_async_copy(chunk_src(c), pbuf, sem).wait()

                @pl.when(c < nch - 1)
                def _prefetch():
                    pltpu.async_copy(chunk_src(c + 1), nxt, nsem)

                process(pbuf, packed)
            return carry
        lax.fori_loop(0, nch // 2, pair_body, 0)

        if nch % 2:
            pltpu.make_async_copy(chunk_src(nch - 1), pbuf0, sem0).wait()
            process(pbuf0, packed)

        pltpu.async_copy(acc, acc_dst(col), asem)
    pltpu.make_async_copy(acc, acc_dst(len(rows) - 1), asem).wait()


def _tc2_body(part_ref, pbt_ref, out_ref):
    part = part_ref[...]
    d = pbt_ref.shape[0]
    cols = []
    for p in range(d // 2):
        wr = part[:, p]
        cols.append(lax.bitcast_convert_type(wr << 16, jnp.float32))
        cols.append(lax.bitcast_convert_type(wr & HI_MASK, jnp.float32))
    if d % 2:
        cols.append(lax.bitcast_convert_type(part[:, d // 2], jnp.float32))
    m = jnp.concatenate([jnp.max(c, axis=0)[None] for c in cols], axis=0)
    out_ref[...] = jnp.where(m > THRESH, m + pbt_ref[...], 0.0)


def kernel(x, edge_index, edge_f, edge_attr, W, b):
    del edge_f, edge_attr
    n, d = x.shape
    e = edge_index.shape[1]
    npad = ((n + 127) // 128) * 128
    assert n < (1 << 16) and npad % (LANES * 8) == 0 and e % NW == 0
    epw = e // NW
    ch = 4000 if epw % 4000 == 0 else epw
    assert ch % (LANES * UNROLL) == 0 and epw % ch == 0
    nrows = d // 2 + d % 2
    sc_rows = tuple((True, _SENT_PACKED) for _ in range(d // 2)) + \
        (((False, _SENT_F32),) if d % 2 else ())
    units = npad // 128
    u = max(v for v in range(1, min(units, 32) + 1) if units % v == 0)
    nb = u * 128
    eu = e // 128
    ue = max(v for v in range(1, min(eu, 1024) + 1) if eu % v == 0)
    eb = ue * 128

    edge_index = edge_index.astype(jnp.int32)
    a_mat = W[:d] - W[d:]
    wb_mat = W[d:]
    xtp = jnp.pad(x.T, ((0, 0), (0, npad - n)))

    pbt, qt = pl.pallas_call(
        _tc1_body,
        grid=(npad // nb,),
        in_specs=[
            pl.BlockSpec((d, nb), lambda i: (0, i)),
            pl.BlockSpec((d, d), lambda i: (0, 0)),
            pl.BlockSpec((d, d), lambda i: (0, 0)),
            pl.BlockSpec((d,), lambda i: (0,)),
        ],
        out_specs=[
            pl.BlockSpec((d, nb), lambda i: (0, i)),
            pl.BlockSpec((nrows, nb), lambda i: (0, i)),
        ],
        out_shape=[
            jax.ShapeDtypeStruct((d, npad), jnp.float32),
            jax.ShapeDtypeStruct((nrows, npad), jnp.int32),
        ],
    )(xtp, a_mat, wb_mat, b)

    packed = pl.pallas_call(
        _pack_body,
        grid=(e // eb,),
        in_specs=[pl.BlockSpec((2, eb), lambda i: (0, i))],
        out_specs=pl.BlockSpec((eb,), lambda i: (i,)),
        out_shape=jax.ShapeDtypeStruct((e,), jnp.int32),
    )(edge_index)

    mesh = plsc.VectorSubcoreMesh(
        core_axis_name="c", subcore_axis_name="s",
        num_cores=NC, num_subcores=NS)
    partial = pl.kernel(
        functools.partial(_sc_body, npad, e, sc_rows, ch),
        out_type=jax.ShapeDtypeStruct((NW * nrows * npad,), jnp.int32),
        mesh=mesh,
        compiler_params=pltpu.CompilerParams(needs_layout_passes=False),
        scratch_types=[
            pltpu.VMEM((npad,), jnp.int32),
            pltpu.VMEM((npad,), jnp.int32),
            pltpu.VMEM((ch,), jnp.int32),
            pltpu.VMEM((ch,), jnp.int32),
            pltpu.SemaphoreType.DMA,
            pltpu.SemaphoreType.DMA,
            pltpu.SemaphoreType.DMA,
            pltpu.SemaphoreType.DMA,
        ],
    )(qt.reshape(-1), packed)
    partial = partial.reshape(NW, nrows, npad)

    outt = pl.pallas_call(
        _tc2_body,
        grid=(npad // nb,),
        in_specs=[
            pl.BlockSpec((NW, nrows, nb), lambda i: (0, 0, i)),
            pl.BlockSpec((d, nb), lambda i: (0, i)),
        ],
        out_specs=pl.BlockSpec((d, nb), lambda i: (0, i)),
        out_shape=jax.ShapeDtypeStruct((d, npad), jnp.float32),
    )(partial, pbt)
    return outt[:, :n].T


# speedup vs baseline: 1.1134x; 1.0188x over previous
import functools

import jax
import jax.numpy as jnp
import numpy as np
from jax import lax
from jax.experimental import pallas as pl
from jax.experimental.pallas import tpu as pltpu
from jax.experimental.pallas import tpu_sc as plsc

NC = 2
NS = 16
LANES = 16
NW = NC * NS
UNROLL = 10

SENTINEL = -3.0e38
THRESH = -1.0e37
HI_MASK = -65536
LO_MASK = 0xFFFF

_F32_BITS = int(np.float32(SENTINEL).view(np.uint32))


def _bf16_bits(u):
    return ((u + 0x7FFF + ((u >> 16) & 1)) >> 16) & 0xFFFF


_SENT_BF = _bf16_bits(_F32_BITS)
_SENT_PACKED = np.int32(np.uint32((_SENT_BF << 16) | _SENT_BF)).item()
_SENT_F32 = np.int32(np.uint32(_F32_BITS)).item()


def _tc1_body(xt_ref, a_ref, wb_ref, b_ref, pbt_ref, qt_ref):
    xt = xt_ref[...]
    d = xt.shape[0]
    dn0 = (((0,), (0,)), ((), ()))
    pbt = lax.dot_general(a_ref[...], xt, dn0,
                          preferred_element_type=jnp.float32,
                          precision=lax.Precision.HIGHEST)
    pbt_ref[...] = pbt + b_ref[...].reshape(d, 1)
    q = lax.dot_general(wb_ref[...], xt, dn0,
                        preferred_element_type=jnp.float32,
                        precision=lax.Precision.HIGHEST)
    qb = lax.bitcast_convert_type(q.astype(jnp.bfloat16),
                                  jnp.uint16).astype(jnp.uint32)
    rows = []
    for p in range(d // 2):
        rows.append(((qb[2 * p + 1] << 16) | qb[2 * p])[None])
    if d % 2:
        rows.append(lax.bitcast_convert_type(q[d - 1], jnp.uint32)[None])
    qt_ref[...] = jnp.concatenate(rows, axis=0).astype(jnp.int32)


def _pack_body(ei_ref, pk_ref):
    ei = ei_ref[...]
    s = ei[0].astype(jnp.uint32)
    dd = ei[1].astype(jnp.uint32)
    pk_ref[...] = ((dd << 16) | s).astype(jnp.int32)


def _extract(pk):
    sv = pk & LO_MASK
    dv = lax.shift_right_logical(pk, 16)
    return sv, dv


def _pend_val(qw, aw, packed):
    if packed:
        q0 = lax.bitcast_convert_type(qw << 16, jnp.float32)
        q1 = lax.bitcast_convert_type(qw & HI_MASK, jnp.float32)
        a0 = lax.bitcast_convert_type(aw << 16, jnp.float32)
        a1 = lax.bitcast_convert_type(aw & HI_MASK, jnp.float32)
        m0 = q0 > a0
        m1 = q1 > a1
        val = (jnp.where(m1, qw, aw) & HI_MASK) | \
              (jnp.where(m0, qw, aw) & LO_MASK)
        return m0 | m1, val
    q = lax.bitcast_convert_type(qw, jnp.float32)
    a = lax.bitcast_convert_type(aw, jnp.float32)
    return q > a, qw


def _sc_body(n, e, rows, ch, qt_hbm, pk_hbm, part_hbm,
             qcol, acc, pbuf0, pbuf1, sem0, sem1, qsem, asem):
    epw = e // NW
    nch = epw // ch
    cid = lax.axis_index("c")
    sid = lax.axis_index("s")
    w = sid * NC + cid
    e0 = w * epw
    nu = ch // (LANES * UNROLL)
    zero16 = jnp.zeros((LANES,), jnp.int32)
    one16 = jnp.ones((LANES,), jnp.int32)
    bufs = ((pbuf0, sem0), (pbuf1, sem1))

    def chunk_src(c):
        return pk_hbm.at[pl.ds(e0 + c * ch, ch)]

    def acc_dst(col):
        return part_hbm.at[pl.ds((w * len(rows) + col) * n, n)]

    def process(pbuf, packed):
        def blk_body(i, vacc):
            base = i * (LANES * UNROLL)
            dvs, qws = [], []
            for j in range(UNROLL):
                pk = pbuf[pl.ds(base + j * LANES, LANES)]
                sv, dv = _extract(pk)
                dvs.append(dv)
                qws.append(plsc.load_gather(qcol, [sv]))
            aws = [plsc.load_gather(acc, [dv]) for dv in dvs]
            for j in range(UNROLL):
                pend, val = _pend_val(qws[j], aws[j], packed)
                plsc.store_scatter(acc, [dvs[j]], val, mask=pend)
            a1s = [plsc.load_gather(acc, [dv]) for dv in dvs]
            for j in range(UNROLL):
                pend, val = _pend_val(qws[j], a1s[j], packed)
                plsc.store_scatter(acc, [dvs[j]], val, mask=pend)
            viol = None
            for j in range(UNROLL):
                a2 = plsc.load_gather(acc, [dvs[j]])
                v, _ = _pend_val(qws[j], a2, packed)
                viol = v if viol is None else (viol | v)
            return vacc | jnp.where(viol, one16, zero16)

        vacc = lax.fori_loop(0, nu, blk_body, zero16)

        @pl.when(jnp.max(vacc) > 0)
        def _deep_repair():
            def grp(g, c2):
                pk = pbuf[pl.ds(g * LANES, LANES)]
                sv, dv = _extract(pk)
                qw = plsc.load_gather(qcol, [sv])

                def rnd(r, c3):
                    aw = plsc.load_gather(acc, [dv])
                    pend, val = _pend_val(qw, aw, packed)
                    plsc.store_scatter(acc, [dv], val, mask=pend)
                    return c3
                lax.fori_loop(0, LANES, rnd, 0)
                return c2
            lax.fori_loop(0, ch // LANES, grp, 0)

    for col, (packed, sent_word) in enumerate(rows):
        pltpu.async_copy(qt_hbm.at[pl.ds(col * n, n)], qcol, qsem)
        pltpu.async_copy(chunk_src(0), pbuf0, sem0)
        if col > 0:
            pltpu.make_async_copy(acc, acc_dst(col - 1), asem).wait()
        pltpu.make_async_copy(qt_hbm.at[pl.ds(col * n, n)], qcol, qsem).wait()

        sent16 = jnp.full((LANES,), sent_word, jnp.int32)

        def init_body(i, carry):
            for j in range(8):
                acc[pl.ds((i * 8 + j) * LANES, LANES)] = sent16
            return carry
        lax.fori_loop(0, n // (LANES * 8), init_body, 0)

        def pair_body(k, carry):
            for bsel in range(2):
                pbuf, sem = bufs[bsel]
                nxt, nsem = bufs[1 - bsel]
                c = k * 2 + bsel
                pltpu.make_async_copy(chunk_src(c), pbuf, sem).wait()

                @pl.when(c < nch - 1)
                def _prefetch():
                    pltpu.async_copy(chunk_src(c + 1), nxt, nsem)

                process(pbuf, packed)
            return carry
        lax.fori_loop(0, nch // 2, pair_body, 0)

        if nch % 2:
            pltpu.make_async_copy(chunk_src(nch - 1), pbuf0, sem0).wait()
            process(pbuf0, packed)

        pltpu.async_copy(acc, acc_dst(col), asem)
    pltpu.make_async_copy(acc, acc_dst(len(rows) - 1), asem).wait()


def _tc2_body(part_ref, pbt_ref, out_ref):
    d = pbt_ref.shape[0]
    nrows = part_ref.shape[0] // NW
    part = part_ref[...].reshape(NW, nrows, part_ref.shape[1])
    cols = []
    for p in range(d // 2):
        wr = part[:, p]
        cols.append(lax.bitcast_convert_type(wr << 16, jnp.float32))
        cols.append(lax.bitcast_convert_type(wr & HI_MASK, jnp.float32))
    if d % 2:
        cols.append(lax.bitcast_convert_type(part[:, d // 2], jnp.float32))
    m = jnp.concatenate([jnp.max(c, axis=0)[None] for c in cols], axis=0)
    out_ref[...] = jnp.where(m > THRESH, m + pbt_ref[...], 0.0)


def kernel(x, edge_index, edge_f, edge_attr, W, b):
    del edge_f, edge_attr
    n, d = x.shape
    e = edge_index.shape[1]
    npad = ((n + 127) // 128) * 128
    assert n < (1 << 16) and npad % (LANES * 8) == 0 and e % NW == 0
    epw = e // NW
    ch = 4000 if epw % 4000 == 0 else epw
    assert ch % (LANES * UNROLL) == 0 and epw % ch == 0
    nrows = d // 2 + d % 2
    sc_rows = tuple((True, _SENT_PACKED) for _ in range(d // 2)) + \
        (((False, _SENT_F32),) if d % 2 else ())
    units = npad // 128
    u = max(v for v in range(1, min(units, 32) + 1) if units % v == 0)
    nb = u * 128
    eu = e // 128
    ue = max(v for v in range(1, min(eu, 1024) + 1) if eu % v == 0)
    eb = ue * 128

    edge_index = edge_index.astype(jnp.int32)
    a_mat = W[:d] - W[d:]
    wb_mat = W[d:]
    xtp = jnp.pad(x.T, ((0, 0), (0, npad - n)))

    pbt, qt = pl.pallas_call(
        _tc1_body,
        grid=(npad // nb,),
        in_specs=[
            pl.BlockSpec((d, nb), lambda i: (0, i)),
            pl.BlockSpec((d, d), lambda i: (0, 0)),
            pl.BlockSpec((d, d), lambda i: (0, 0)),
            pl.BlockSpec((d,), lambda i: (0,)),
        ],
        out_specs=[
            pl.BlockSpec((d, nb), lambda i: (0, i)),
            pl.BlockSpec((nrows, nb), lambda i: (0, i)),
        ],
        out_shape=[
            jax.ShapeDtypeStruct((d, npad), jnp.float32),
            jax.ShapeDtypeStruct((nrows, npad), jnp.int32),
        ],
    )(xtp, a_mat, wb_mat, b)

    packed = pl.pallas_call(
        _pack_body,
        grid=(e // eb,),
        in_specs=[pl.BlockSpec((2, eb), lambda i: (0, i))],
        out_specs=pl.BlockSpec((eb,), lambda i: (i,)),
        out_shape=jax.ShapeDtypeStruct((e,), jnp.int32),
    )(edge_index)

    mesh = plsc.VectorSubcoreMesh(
        core_axis_name="c", subcore_axis_name="s",
        num_cores=NC, num_subcores=NS)
    partial = pl.kernel(
        functools.partial(_sc_body, npad, e, sc_rows, ch),
        out_type=jax.ShapeDtypeStruct((NW * nrows * npad,), jnp.int32),
        mesh=mesh,
        compiler_params=pltpu.CompilerParams(needs_layout_passes=False),
        scratch_types=[
            pltpu.VMEM((npad,), jnp.int32),
            pltpu.VMEM((npad,), jnp.int32),
            pltpu.VMEM((ch,), jnp.int32),
            pltpu.VMEM((ch,), jnp.int32),
            pltpu.SemaphoreType.DMA,
            pltpu.SemaphoreType.DMA,
            pltpu.SemaphoreType.DMA,
            pltpu.SemaphoreType.DMA,
        ],
    )(qt.reshape(-1), packed)
    partial = partial.reshape(NW * nrows, npad)

    outt = pl.pallas_call(
        _tc2_body,
        grid=(npad // nb,),
        in_specs=[
            pl.BlockSpec((NW * nrows, nb), lambda i: (0, i)),
            pl.BlockSpec((d, nb), lambda i: (0, i)),
        ],
        out_specs=pl.BlockSpec((d, nb), lambda i: (0, i)),
        out_shape=jax.ShapeDtypeStruct((d, npad), jnp.float32),
    )(partial, pbt)
    return outt[:, :n].T


# speedup vs baseline: 1.2907x; 1.1592x over previous
import functools

import jax
import jax.numpy as jnp
import numpy as np
from jax import lax
from jax.experimental import pallas as pl
from jax.experimental.pallas import tpu as pltpu
from jax.experimental.pallas import tpu_sc as plsc

NC = 2
NS = 16
LANES = 16
NW = NC * NS
UNROLL = 5

SENTINEL = -3.0e38
THRESH = -1.0e37
HI_MASK = -65536
LO_MASK = 0xFFFF

_F32_BITS = int(np.float32(SENTINEL).view(np.uint32))


def _bf16_bits(u):
    return ((u + 0x7FFF + ((u >> 16) & 1)) >> 16) & 0xFFFF


_SENT_BF = _bf16_bits(_F32_BITS)
_SENT_PACKED = np.int32(np.uint32((_SENT_BF << 16) | _SENT_BF)).item()
_SENT_F32 = np.int32(np.uint32(_F32_BITS)).item()


def _tc1_body(xt_ref, a_ref, wb_ref, b_ref, pbt_ref, qt_ref):
    xt = xt_ref[...]
    d = xt.shape[0]
    dn0 = (((0,), (0,)), ((), ()))
    pbt = lax.dot_general(a_ref[...], xt, dn0,
                          preferred_element_type=jnp.float32,
                          precision=lax.Precision.HIGHEST)
    pbt_ref[...] = pbt + b_ref[...].reshape(d, 1)
    q = lax.dot_general(wb_ref[...], xt, dn0,
                        preferred_element_type=jnp.float32,
                        precision=lax.Precision.HIGHEST)
    qb = lax.bitcast_convert_type(q.astype(jnp.bfloat16),
                                  jnp.uint16).astype(jnp.uint32)
    rows = []
    for p in range(d // 2):
        rows.append(((qb[2 * p + 1] << 16) | qb[2 * p])[None])
    if d % 2:
        rows.append(lax.bitcast_convert_type(q[d - 1], jnp.uint32)[None])
    qt_ref[...] = jnp.concatenate(rows, axis=0).astype(jnp.int32)


def _pack_body(ei_ref, pk_ref):
    ei = ei_ref[...]
    s = ei[0].astype(jnp.uint32)
    dd = ei[1].astype(jnp.uint32)
    pk_ref[...] = ((dd << 16) | s).astype(jnp.int32)


def _extract(pk):
    sv = pk & LO_MASK
    dv = lax.shift_right_logical(pk, 16)
    return sv, dv


def _pend_val(qw, aw, packed):
    if packed:
        q0 = lax.bitcast_convert_type(qw << 16, jnp.float32)
        q1 = lax.bitcast_convert_type(qw & HI_MASK, jnp.float32)
        a0 = lax.bitcast_convert_type(aw << 16, jnp.float32)
        a1 = lax.bitcast_convert_type(aw & HI_MASK, jnp.float32)
        m0 = q0 > a0
        m1 = q1 > a1
        val = (jnp.where(m1, qw, aw) & HI_MASK) | \
              (jnp.where(m0, qw, aw) & LO_MASK)
        return m0 | m1, val
    q = lax.bitcast_convert_type(qw, jnp.float32)
    a = lax.bitcast_convert_type(aw, jnp.float32)
    return q > a, qw


def _sc_body(n, e, rows, ch, qt_hbm, pk_hbm, part_hbm,
             qcol, acc, pbuf0, pbuf1, sem0, sem1, qsem, asem):
    epw = e // NW
    nch = epw // ch
    cid = lax.axis_index("c")
    sid = lax.axis_index("s")
    w = sid * NC + cid
    e0 = w * epw
    nu = ch // (LANES * UNROLL)
    zero16 = jnp.zeros((LANES,), jnp.int32)
    one16 = jnp.ones((LANES,), jnp.int32)
    bufs = ((pbuf0, sem0), (pbuf1, sem1))

    def chunk_src(c):
        return pk_hbm.at[pl.ds(e0 + c * ch, ch)]

    def acc_dst(col):
        return part_hbm.at[pl.ds((w * len(rows) + col) * n, n)]

    def process(pbuf, packed):
        def blk_body(i, vacc):
            base = i * (LANES * UNROLL)
            dvs, qws = [], []
            for j in range(UNROLL):
                pk = pbuf[pl.ds(base + j * LANES, LANES)]
                sv, dv = _extract(pk)
                dvs.append(dv)
                qws.append(plsc.load_gather(qcol, [sv]))
            aws = [plsc.load_gather(acc, [dv]) for dv in dvs]
            for j in range(UNROLL):
                pend, val = _pend_val(qws[j], aws[j], packed)
                plsc.store_scatter(acc, [dvs[j]], val, mask=pend)
            a1s = [plsc.load_gather(acc, [dv]) for dv in dvs]
            for j in range(UNROLL):
                pend, val = _pend_val(qws[j], a1s[j], packed)
                plsc.store_scatter(acc, [dvs[j]], val, mask=pend)
            viol = None
            for j in range(UNROLL):
                a2 = plsc.load_gather(acc, [dvs[j]])
                v, _ = _pend_val(qws[j], a2, packed)
                viol = v if viol is None else (viol | v)
            return vacc | jnp.where(viol, one16, zero16)

        vacc = lax.fori_loop(0, nu, blk_body, zero16)

        @pl.when(jnp.max(vacc) > 0)
        def _deep_repair():
            def grp(g, c2):
                pk = pbuf[pl.ds(g * LANES, LANES)]
                sv, dv = _extract(pk)
                qw = plsc.load_gather(qcol, [sv])

                def rnd(r, c3):
                    aw = plsc.load_gather(acc, [dv])
                    pend, val = _pend_val(qw, aw, packed)
                    plsc.store_scatter(acc, [dv], val, mask=pend)
                    return c3
                lax.fori_loop(0, LANES, rnd, 0)
                return c2
            lax.fori_loop(0, ch // LANES, grp, 0)

    for col, (packed, sent_word) in enumerate(rows):
        pltpu.async_copy(qt_hbm.at[pl.ds(col * n, n)], qcol, qsem)
        pltpu.async_copy(chunk_src(0), pbuf0, sem0)
        if col > 0:
            pltpu.make_async_copy(acc, acc_dst(col - 1), asem).wait()

        sent16 = jnp.full((LANES,), sent_word, jnp.int32)

        def init_body(i, carry):
            for j in range(8):
                acc[pl.ds((i * 8 + j) * LANES, LANES)] = sent16
            return carry
        lax.fori_loop(0, n // (LANES * 8), init_body, 0)
        pltpu.make_async_copy(qt_hbm.at[pl.ds(col * n, n)], qcol, qsem).wait()

        def pair_body(k, carry):
            for bsel in range(2):
                pbuf, sem = bufs[bsel]
                nxt, nsem = bufs[1 - bsel]
                c = k * 2 + bsel
                pltpu.make_async_copy(chunk_src(c), pbuf, sem).wait()

                @pl.when(c < nch - 1)
                def _prefetch():
                    pltpu.async_copy(chunk_src(c + 1), nxt, nsem)

                process(pbuf, packed)
            return carry
        lax.fori_loop(0, nch // 2, pair_body, 0)

        if nch % 2:
            pltpu.make_async_copy(chunk_src(nch - 1), pbuf0, sem0).wait()
            process(pbuf0, packed)

        pltpu.async_copy(acc, acc_dst(col), asem)
    pltpu.make_async_copy(acc, acc_dst(len(rows) - 1), asem).wait()


def _tc2_body(part_ref, pbt_ref, out_ref):
    d = pbt_ref.shape[0]
    nrows = part_ref.shape[0] // NW
    part = part_ref[...].reshape(NW, nrows, part_ref.shape[1])
    cols = []
    for p in range(d // 2):
        wr = part[:, p]
        cols.append(lax.bitcast_convert_type(wr << 16, jnp.float32))
        cols.append(lax.bitcast_convert_type(wr & HI_MASK, jnp.float32))
    if d % 2:
        cols.append(lax.bitcast_convert_type(part[:, d // 2], jnp.float32))
    m = jnp.concatenate([jnp.max(c, axis=0)[None] for c in cols], axis=0)
    out_ref[...] = jnp.where(m > THRESH, m + pbt_ref[...], 0.0)


def kernel(x, edge_index, edge_f, edge_attr, W, b):
    del edge_f, edge_attr
    n, d = x.shape
    e = edge_index.shape[1]
    npad = ((n + 127) // 128) * 128
    assert n < (1 << 16) and npad % (LANES * 8) == 0 and e % NW == 0
    epw = e // NW
    ch = 4000 if epw % 4000 == 0 else epw
    assert ch % (LANES * UNROLL) == 0 and epw % ch == 0
    nrows = d // 2 + d % 2
    sc_rows = tuple((True, _SENT_PACKED) for _ in range(d // 2)) + \
        (((False, _SENT_F32),) if d % 2 else ())
    units = npad // 128
    u = max(v for v in range(1, min(units, 32) + 1) if units % v == 0)
    nb = u * 128
    eu = e // 128
    ue = max(v for v in range(1, min(eu, 1024) + 1) if eu % v == 0)
    eb = ue * 128

    edge_index = edge_index.astype(jnp.int32)
    a_mat = W[:d] - W[d:]
    wb_mat = W[d:]
    xtp = jnp.pad(x.T, ((0, 0), (0, npad - n)))

    pbt, qt = pl.pallas_call(
        _tc1_body,
        grid=(npad // nb,),
        in_specs=[
            pl.BlockSpec((d, nb), lambda i: (0, i)),
            pl.BlockSpec((d, d), lambda i: (0, 0)),
            pl.BlockSpec((d, d), lambda i: (0, 0)),
            pl.BlockSpec((d,), lambda i: (0,)),
        ],
        out_specs=[
            pl.BlockSpec((d, nb), lambda i: (0, i)),
            pl.BlockSpec((nrows, nb), lambda i: (0, i)),
        ],
        out_shape=[
            jax.ShapeDtypeStruct((d, npad), jnp.float32),
            jax.ShapeDtypeStruct((nrows, npad), jnp.int32),
        ],
    )(xtp, a_mat, wb_mat, b)

    packed = pl.pallas_call(
        _pack_body,
        grid=(e // eb,),
        in_specs=[pl.BlockSpec((2, eb), lambda i: (0, i))],
        out_specs=pl.BlockSpec((eb,), lambda i: (i,)),
        out_shape=jax.ShapeDtypeStruct((e,), jnp.int32),
    )(edge_index)

    mesh = plsc.VectorSubcoreMesh(
        core_axis_name="c", subcore_axis_name="s",
        num_cores=NC, num_subcores=NS)
    partial = pl.kernel(
        functools.partial(_sc_body, npad, e, sc_rows, ch),
        out_type=jax.ShapeDtypeStruct((NW * nrows * npad,), jnp.int32),
        mesh=mesh,
        compiler_params=pltpu.CompilerParams(needs_layout_passes=False),
        scratch_types=[
            pltpu.VMEM((npad,), jnp.int32),
            pltpu.VMEM((npad,), jnp.int32),
            pltpu.VMEM((ch,), jnp.int32),
            pltpu.VMEM((ch,), jnp.int32),
            pltpu.SemaphoreType.DMA,
            pltpu.SemaphoreType.DMA,
            pltpu.SemaphoreType.DMA,
            pltpu.SemaphoreType.DMA,
        ],
    )(qt.reshape(-1), packed)
    partial = partial.reshape(NW * nrows, npad)

    outt = pl.pallas_call(
        _tc2_body,
        grid=(npad // nb,),
        in_specs=[
            pl.BlockSpec((NW * nrows, nb), lambda i: (0, i)),
            pl.BlockSpec((d, nb), lambda i: (0, i)),
        ],
        out_specs=pl.BlockSpec((d, nb), lambda i: (0, i)),
        out_shape=jax.ShapeDtypeStruct((d, npad), jnp.float32),
    )(partial, pbt)
    return outt[:, :n].T


# speedup vs baseline: 1.3005x; 1.0076x over previous
import functools

import jax
import jax.numpy as jnp
import numpy as np
from jax import lax
from jax.experimental import pallas as pl
from jax.experimental.pallas import tpu as pltpu
from jax.experimental.pallas import tpu_sc as plsc

NC = 2
NS = 16
LANES = 16
NW = NC * NS
UNROLL = 5

SENTINEL = -3.0e38
THRESH = -1.0e37
HI_MASK = -65536
LO_MASK = 0xFFFF

_F32_BITS = int(np.float32(SENTINEL).view(np.uint32))


def _bf16_bits(u):
    return ((u + 0x7FFF + ((u >> 16) & 1)) >> 16) & 0xFFFF


_SENT_BF = _bf16_bits(_F32_BITS)
_SENT_PACKED = np.int32(np.uint32((_SENT_BF << 16) | _SENT_BF)).item()
_SENT_F32 = np.int32(np.uint32(_F32_BITS)).item()


def _tc1_body(xt_ref, a_ref, wb_ref, b_ref, pbt_ref, qt_ref):
    xt = xt_ref[...]
    d = xt.shape[0]
    dn0 = (((0,), (0,)), ((), ()))
    pbt = lax.dot_general(a_ref[...], xt, dn0,
                          preferred_element_type=jnp.float32,
                          precision=lax.Precision.HIGHEST)
    pbt_ref[...] = pbt + b_ref[...].reshape(d, 1)
    q = lax.dot_general(wb_ref[...], xt, dn0,
                        preferred_element_type=jnp.float32,
                        precision=lax.Precision.HIGHEST)
    qb = lax.bitcast_convert_type(q.astype(jnp.bfloat16),
                                  jnp.uint16).astype(jnp.uint32)
    rows = []
    for p in range(d // 2):
        rows.append(((qb[2 * p + 1] << 16) | qb[2 * p])[None])
    if d % 2:
        rows.append(lax.bitcast_convert_type(q[d - 1], jnp.uint32)[None])
    qt_ref[...] = jnp.concatenate(rows, axis=0).astype(jnp.int32)


def _pack_body(ei_ref, pk_ref):
    ei = ei_ref[...]
    s = ei[0].astype(jnp.uint32)
    dd = ei[1].astype(jnp.uint32)
    pk_ref[...] = ((dd << 16) | s).astype(jnp.int32)


def _extract(pk):
    sv = pk & LO_MASK
    dv = lax.shift_right_logical(pk, 16)
    return sv, dv


def _pend_val(qw, aw, packed):
    if packed:
        q0 = lax.bitcast_convert_type(qw << 16, jnp.float32)
        q1 = lax.bitcast_convert_type(qw & HI_MASK, jnp.float32)
        a0 = lax.bitcast_convert_type(aw << 16, jnp.float32)
        a1 = lax.bitcast_convert_type(aw & HI_MASK, jnp.float32)
        m0 = q0 > a0
        m1 = q1 > a1
        val = (jnp.where(m1, qw, aw) & HI_MASK) | \
              (jnp.where(m0, qw, aw) & LO_MASK)
        return m0 | m1, val
    q = lax.bitcast_convert_type(qw, jnp.float32)
    a = lax.bitcast_convert_type(aw, jnp.float32)
    return q > a, qw


def _sc_body(n, e, rows, ch, qt_hbm, pk_hbm, part_hbm,
             qcol, acc, pbuf0, pbuf1, sem0, sem1, qsem, asem):
    epw = e // NW
    nch = epw // ch
    cid = lax.axis_index("c")
    sid = lax.axis_index("s")
    w = sid * NC + cid
    e0 = w * epw
    nu = ch // (LANES * UNROLL)
    zero16 = jnp.zeros((LANES,), jnp.int32)
    one16 = jnp.ones((LANES,), jnp.int32)
    bufs = ((pbuf0, sem0), (pbuf1, sem1))

    def chunk_src(c):
        return pk_hbm.at[pl.ds(e0 + c * ch, ch)]

    def acc_dst(col):
        return part_hbm.at[pl.ds((w * len(rows) + col) * n, n)]

    def process(pbuf, packed):
        def blk_body(i, vacc):
            base = i * (LANES * UNROLL)
            dvs, qws = [], []
            for j in range(UNROLL):
                pk = pbuf[pl.ds(base + j * LANES, LANES)]
                sv, dv = _extract(pk)
                dvs.append(dv)
                qws.append(plsc.load_gather(qcol, [sv]))
            aws = [plsc.load_gather(acc, [dv]) for dv in dvs]
            for j in range(UNROLL):
                pend, val = _pend_val(qws[j], aws[j], packed)
                plsc.store_scatter(acc, [dvs[j]], val, mask=pend)
            a1s = [plsc.load_gather(acc, [dv]) for dv in dvs]
            for j in range(UNROLL):
                pend, val = _pend_val(qws[j], a1s[j], packed)
                plsc.store_scatter(acc, [dvs[j]], val, mask=pend)
            viol = None
            for j in range(UNROLL):
                a2 = plsc.load_gather(acc, [dvs[j]])
                v, _ = _pend_val(qws[j], a2, packed)
                viol = v if viol is None else (viol | v)
            return vacc | jnp.where(viol, one16, zero16)

        vacc = lax.fori_loop(0, nu, blk_body, zero16)

        @pl.when(jnp.max(vacc) > 0)
        def _deep_repair():
            def grp(g, c2):
                pk = pbuf[pl.ds(g * LANES, LANES)]
                sv, dv = _extract(pk)
                qw = plsc.load_gather(qcol, [sv])

                def rnd(r, c3):
                    aw = plsc.load_gather(acc, [dv])
                    pend, val = _pend_val(qw, aw, packed)
                    plsc.store_scatter(acc, [dv], val, mask=pend)
                    return c3
                lax.fori_loop(0, LANES, rnd, 0)
                return c2
            lax.fori_loop(0, ch // LANES, grp, 0)

    for col, (packed, sent_word) in enumerate(rows):
        pltpu.async_copy(qt_hbm.at[pl.ds(col * n, n)], qcol, qsem)
        pltpu.async_copy(chunk_src(0), pbuf0, sem0)
        if col > 0:
            pltpu.make_async_copy(acc, acc_dst(col - 1), asem).wait()

        sent16 = jnp.full((LANES,), sent_word, jnp.int32)

        def init_body(i, carry):
            for j in range(8):
                acc[pl.ds((i * 8 + j) * LANES, LANES)] = sent16
            return carry
        lax.fori_loop(0, n // (LANES * 8), init_body, 0)
        pltpu.make_async_copy(qt_hbm.at[pl.ds(col * n, n)], qcol, qsem).wait()

        def pair_body(k, carry):
            for bsel in range(2):
                pbuf, sem = bufs[bsel]
                nxt, nsem = bufs[1 - bsel]
                c = k * 2 + bsel
                pltpu.make_async_copy(chunk_src(c), pbuf, sem).wait()

                @pl.when(c < nch - 1)
                def _prefetch():
                    pltpu.async_copy(chunk_src(c + 1), nxt, nsem)

                process(pbuf, packed)
            return carry
        lax.fori_loop(0, nch // 2, pair_body, 0)

        if nch % 2:
            pltpu.make_async_copy(chunk_src(nch - 1), pbuf0, sem0).wait()
            process(pbuf0, packed)

        pltpu.async_copy(acc, acc_dst(col), asem)
    pltpu.make_async_copy(acc, acc_dst(len(rows) - 1), asem).wait()


def _tc2_body(part_ref, pbt_ref, out_ref):
    d = pbt_ref.shape[0]
    nrows = part_ref.shape[0] // NW
    part = part_ref[...].reshape(NW, nrows, part_ref.shape[1])
    cols = []
    for p in range(d // 2):
        wr = part[:, p]
        cols.append(lax.bitcast_convert_type(wr << 16, jnp.float32))
        cols.append(lax.bitcast_convert_type(wr & HI_MASK, jnp.float32))
    if d % 2:
        cols.append(lax.bitcast_convert_type(part[:, d // 2], jnp.float32))
    m = jnp.concatenate([jnp.max(c, axis=0)[None] for c in cols], axis=0)
    out_ref[...] = jnp.where(m > THRESH, m + pbt_ref[...], 0.0)


def kernel(x, edge_index, edge_f, edge_attr, W, b):
    del edge_f, edge_attr
    n, d = x.shape
    e = edge_index.shape[1]
    npad = ((n + 127) // 128) * 128
    assert n < (1 << 16) and npad % (LANES * 8) == 0 and e % NW == 0
    epw = e // NW
    ch = 10000 if epw % 10000 == 0 else epw
    assert ch % (LANES * UNROLL) == 0 and epw % ch == 0
    nrows = d // 2 + d % 2
    sc_rows = tuple((True, _SENT_PACKED) for _ in range(d // 2)) + \
        (((False, _SENT_F32),) if d % 2 else ())
    units = npad // 128
    u = max(v for v in range(1, min(units, 32) + 1) if units % v == 0)
    nb = u * 128
    eu = e // 128
    ue = max(v for v in range(1, min(eu, 1024) + 1) if eu % v == 0)
    eb = ue * 128

    edge_index = edge_index.astype(jnp.int32)
    a_mat = W[:d] - W[d:]
    wb_mat = W[d:]
    xtp = jnp.pad(x.T, ((0, 0), (0, npad - n)))

    pbt, qt = pl.pallas_call(
        _tc1_body,
        grid=(npad // nb,),
        in_specs=[
            pl.BlockSpec((d, nb), lambda i: (0, i)),
            pl.BlockSpec((d, d), lambda i: (0, 0)),
            pl.BlockSpec((d, d), lambda i: (0, 0)),
            pl.BlockSpec((d,), lambda i: (0,)),
        ],
        out_specs=[
            pl.BlockSpec((d, nb), lambda i: (0, i)),
            pl.BlockSpec((nrows, nb), lambda i: (0, i)),
        ],
        out_shape=[
            jax.ShapeDtypeStruct((d, npad), jnp.float32),
            jax.ShapeDtypeStruct((nrows, npad), jnp.int32),
        ],
    )(xtp, a_mat, wb_mat, b)

    packed = pl.pallas_call(
        _pack_body,
        grid=(e // eb,),
        in_specs=[pl.BlockSpec((2, eb), lambda i: (0, i))],
        out_specs=pl.BlockSpec((eb,), lambda i: (i,)),
        out_shape=jax.ShapeDtypeStruct((e,), jnp.int32),
    )(edge_index)

    mesh = plsc.VectorSubcoreMesh(
        core_axis_name="c", subcore_axis_name="s",
        num_cores=NC, num_subcores=NS)
    partial = pl.kernel(
        functools.partial(_sc_body, npad, e, sc_rows, ch),
        out_type=jax.ShapeDtypeStruct((NW * nrows * npad,), jnp.int32),
        mesh=mesh,
        compiler_params=pltpu.CompilerParams(needs_layout_passes=False),
        scratch_types=[
            pltpu.VMEM((npad,), jnp.int32),
            pltpu.VMEM((npad,), jnp.int32),
            pltpu.VMEM((ch,), jnp.int32),
            pltpu.VMEM((ch,), jnp.int32),
            pltpu.SemaphoreType.DMA,
            pltpu.SemaphoreType.DMA,
            pltpu.SemaphoreType.DMA,
            pltpu.SemaphoreType.DMA,
        ],
    )(qt.reshape(-1), packed)
    partial = partial.reshape(NW * nrows, npad)

    outt = pl.pallas_call(
        _tc2_body,
        grid=(npad // nb,),
        in_specs=[
            pl.BlockSpec((NW * nrows, nb), lambda i: (0, i)),
            pl.BlockSpec((d, nb), lambda i: (0, i)),
        ],
        out_specs=pl.BlockSpec((d, nb), lambda i: (0, i)),
        out_shape=jax.ShapeDtypeStruct((d, npad), jnp.float32),
    )(partial, pbt)
    return outt[:, :n].T
